# baseline (speedup 1.0000x reference)
import functools

import jax
import jax.numpy as jnp
from jax import lax
from jax.experimental import pallas as pl
from jax.experimental.pallas import tpu as pltpu
from jax.experimental.pallas import tpu_sc as plsc

N0 = 25000
N1 = 25000
N = N0 + N1
E = 800000
D_IN = 128
HID = 64
COM = 64
H = 4
DH = COM // H
NET = 4
NCLS = 16
NEG = 0.2

_BLK = 1000
_G = 5
_K = _G * 128
_NCH = E // _K
_G1 = 10
_K1 = _G1 * 128
_NCH1 = E // _K1
_NSUB = 16
_NBLK = N // 128
_NTL = N - _NBLK * 128


def _mm_body(x_ref, w_ref, b_ref, o_ref, *, act):
    y = jnp.dot(x_ref[...], w_ref[...], preferred_element_type=jnp.float32)
    y = y + b_ref[...]
    if act == "relu":
        y = jnp.maximum(y, 0.0)
    o_ref[...] = y


def _mm(x, w, b, act=None, blk=_BLK):
    m, k = x.shape
    n = w.shape[1]
    return pl.pallas_call(
        functools.partial(_mm_body, act=act),
        grid=(m // blk,),
        in_specs=[
            pl.BlockSpec((blk, k), lambda i: (i, 0)),
            pl.BlockSpec((k, n), lambda i: (0, 0)),
            pl.BlockSpec((n,), lambda i: (0,)),
        ],
        out_specs=pl.BlockSpec((blk, n), lambda i: (i, 0)),
        out_shape=jax.ShapeDtypeStruct((m, n), jnp.float32),
    )(x, w, b)


def _zsv_body(h_ref, wz_ref, a_ref, zp_ref, sv_ref):
    z = jnp.dot(h_ref[...], wz_ref[...], preferred_element_type=jnp.float32)
    zp_ref[0] = z[:, :32]
    zp_ref[1] = z[:, 32:]
    sv_ref[...] = jnp.dot(z, a_ref[...], preferred_element_type=jnp.float32)


def _zsv(h, wz, amat):
    return pl.pallas_call(
        _zsv_body,
        grid=(N // _BLK,),
        in_specs=[
            pl.BlockSpec((_BLK, HID), lambda i: (i, 0)),
            pl.BlockSpec((HID, COM), lambda i: (0, 0)),
            pl.BlockSpec((COM, 2 * H), lambda i: (0, 0)),
        ],
        out_specs=[
            pl.BlockSpec((2, _BLK, 32), lambda i: (0, i, 0)),
            pl.BlockSpec((_BLK, 2 * H), lambda i: (i, 0)),
        ],
        out_shape=[
            jax.ShapeDtypeStruct((2, N, 32), jnp.float32),
            jax.ShapeDtypeStruct((N, 2 * H), jnp.float32),
        ],
    )(h, wz, amat)


def _ch_body(op_ref, h_ref, wt_ref, bt_ref, bc_ref, com_ref, h2_ref):
    agg = jnp.concatenate([op_ref[0], op_ref[1]], axis=1)
    com_ref[...] = jnp.maximum(agg + bc_ref[...], 0.0)
    t = lax.div(pl.program_id(0), N0 // _BLK)
    y = jnp.dot(h_ref[...], wt_ref[0], preferred_element_type=jnp.float32)
    h2_ref[...] = jnp.maximum(y + bt_ref[pl.ds(t, 1), :], 0.0)


def _ch(outp, h, wt2, bt2, bc0):
    return pl.pallas_call(
        _ch_body,
        grid=(N // _BLK,),
        in_specs=[
            pl.BlockSpec((2, _BLK, 32), lambda i: (0, i, 0)),
            pl.BlockSpec((_BLK, HID), lambda i: (i, 0)),
            pl.BlockSpec((1, HID, HID), lambda i: (lax.div(i, N0 // _BLK), 0, 0)),
            pl.BlockSpec((2, HID), lambda i: (0, 0)),
            pl.BlockSpec((COM,), lambda i: (0,)),
        ],
        out_specs=[
            pl.BlockSpec((_BLK, COM), lambda i: (i, 0)),
            pl.BlockSpec((_BLK, HID), lambda i: (i, 0)),
        ],
        out_shape=[
            jax.ShapeDtypeStruct((N, COM), jnp.float32),
            jax.ShapeDtypeStruct((N, HID), jnp.float32),
        ],
    )(outp, h, wt2, bt2, bc0)


def _fin_body(op_ref, h2_ref, com_ref, wc_ref, bc_ref, wt_ref, bt_ref,
              wp_ref, bp_ref, o_ref):
    agg = jnp.concatenate([op_ref[0], op_ref[1]], axis=1)
    cm = jnp.dot(com_ref[...], wc_ref[...], preferred_element_type=jnp.float32)
    com2 = jnp.maximum(agg + cm + bc_ref[...], 0.0)
    t = lax.div(pl.program_id(0), N0 // _BLK)
    y = jnp.dot(h2_ref[...], wt_ref[0], preferred_element_type=jnp.float32)
    h3 = jnp.maximum(y + bt_ref[pl.ds(t, 1), :], 0.0)
    lg = (jnp.dot(h3, wp_ref[:HID], preferred_element_type=jnp.float32)
          + jnp.dot(com2, wp_ref[HID:], preferred_element_type=jnp.float32)
          + bp_ref[...])
    nrm = jnp.sqrt(jnp.sum(lg * lg, axis=1, keepdims=True))
    o_ref[...] = lg / jnp.maximum(nrm, 1e-12)


def _fin(outp1, h2, com1, wc1, bc1, wt2, bt2, wp, bp):
    return pl.pallas_call(
        _fin_body,
        grid=(N // _BLK,),
        in_specs=[
            pl.BlockSpec((2, _BLK, 32), lambda i: (0, i, 0)),
            pl.BlockSpec((_BLK, HID), lambda i: (i, 0)),
            pl.BlockSpec((_BLK, COM), lambda i: (i, 0)),
            pl.BlockSpec((COM, COM), lambda i: (0, 0)),
            pl.BlockSpec((COM,), lambda i: (0,)),
            pl.BlockSpec((1, HID, HID), lambda i: (lax.div(i, N0 // _BLK), 0, 0)),
            pl.BlockSpec((2, HID), lambda i: (0, 0)),
            pl.BlockSpec((HID + COM, NCLS), lambda i: (0, 0)),
            pl.BlockSpec((NCLS,), lambda i: (0,)),
        ],
        out_specs=pl.BlockSpec((_BLK, NCLS), lambda i: (i, 0)),
        out_shape=jax.ShapeDtypeStruct((N, NCLS), jnp.float32),
    )(outp1, h2, com1, wc1, bc1, wt2, bt2, wp, bp)


def _p1_body(sv_hbm, aet_hbm, src_hbm, dst_hbm, ef_hbm,
             exb, dhb,
             src3, dst3, efv, ssv, sdv, ex3, exw, aetv, sem, den_sh):
    c = lax.axis_index("c")
    w = lax.axis_index("s")
    io = lax.iota(jnp.int32, 16)
    half = lax.shift_right_logical(io, 1)
    par = lax.bitwise_and(io, 1)
    hsel = 2 * c + par
    zeros16 = jnp.zeros((16,), jnp.float32)

    for g in range(_G1):
        def _zw(r, _, g=g):
            exw[g, r, pl.ds(0, 16)] = zeros16
            return _
        lax.fori_loop(0, 128, _zw, None)

    nblk = (_NBLK - w + _NSUB - 1) // _NSUB

    def zblk(i, _):
        r = (w + i * _NSUB) * 128
        pltpu.sync_copy(exw.at[0], den_sh.at[pl.ds(r, 128)])
        return _
    lax.fori_loop(0, nblk, zblk, None)

    @pl.when(w == _NSUB - 1)
    def _zero_tail():
        pltpu.sync_copy(exw.at[0].at[pl.ds(0, _NTL)],
                        den_sh.at[pl.ds(_NBLK * 128, _NTL)])
    pltpu.sync_copy(aet_hbm, aetv)
    plsc.subcore_barrier()

    nch = (_NCH1 - w + _NSUB - 1) // _NSUB

    def p1(ic, _):
        ci = w + ic * _NSUB
        c0 = pltpu.async_copy(src_hbm.at[ci], src3, sem)
        c1 = pltpu.async_copy(dst_hbm.at[ci], dst3, sem)
        c2 = pltpu.async_copy(ef_hbm.at[ci], efv, sem)
        c0.wait(); c1.wait(); c2.wait()
        gcps = []
        for g in range(_G1):
            gcps.append(pltpu.async_copy(sv_hbm.at[src3.at[g]], ssv.at[g], sem))
            gcps.append(pltpu.async_copy(sv_hbm.at[dst3.at[g]], sdv.at[g], sem))
        for cp in gcps:
            cp.wait()
        for g in range(_G1):
            gs = jnp.full((16,), g, jnp.int32)

            def cgrp(j, _, gs=gs, g=g):
                row = j * 8 + half
                s1 = plsc.load_gather(ssv, [gs, row, hsel])
                s2 = plsc.load_gather(sdv, [gs, row, 4 + hsel])
                efl = plsc.load_gather(efv, [g * 128 + row])
                s3 = plsc.load_gather(aetv, [efl, hsel])
                lg = s1 + s2 + s3
                lg = jnp.maximum(lg, NEG * lg)
                ex = jnp.exp(lg)
                plsc.store_scatter(ex3, [gs, row, par], ex)
                plsc.store_scatter(exw, [gs, row, par], ex)
                return _
            lax.fori_loop(0, 16, cgrp, None)
        cpo = pltpu.async_copy(ex3, exb.at[c, ci], sem)
        for g in range(_G1):
            pltpu.sync_copy(exw.at[g], den_sh.at[dst3.at[g]], add=True)
        cpo.wait()
        return _
    lax.fori_loop(0, nch, p1, None)
    plsc.subcore_barrier()

    def dblk(i, _):
        r = (w + i * _NSUB) * 128
        pltpu.sync_copy(den_sh.at[pl.ds(r, 128), pl.ds(0, 8)],
                        dhb.at[c, pl.ds(r, 128)])
        return _
    lax.fori_loop(0, nblk, dblk, None)

    @pl.when(w == _NSUB - 1)
    def _den_tail():
        pltpu.sync_copy(den_sh.at[pl.ds(_NBLK * 128, _NTL), pl.ds(0, 8)],
                        dhb.at[c, pl.ds(_NBLK * 128, _NTL)])


def _p2_body(zp_hbm, exb, dhb, src_hbm, dst_hbm,
             outp,
             src3, dst3, ex3, den3, wbuf, sem, out_sh):
    c = lax.axis_index("c")
    w = lax.axis_index("s")
    io = lax.iota(jnp.int32, 16)
    half = lax.shift_right_logical(io, 1)
    par = lax.bitwise_and(io, 1)
    zeros16 = jnp.zeros((16,), jnp.float32)

    for g in range(2):
        def _zw(r, _, g=g):
            wbuf[g, r, pl.ds(0, 16)] = zeros16
            wbuf[g, r, pl.ds(16, 16)] = zeros16
            return _
        lax.fori_loop(0, 128, _zw, None)

    nblk = (_NBLK - w + _NSUB - 1) // _NSUB

    def zblk(i, _):
        r = (w + i * _NSUB) * 128
        pltpu.sync_copy(wbuf.at[0], out_sh.at[pl.ds(r, 128)])
        return _
    lax.fori_loop(0, nblk, zblk, None)

    @pl.when(w == _NSUB - 1)
    def _zero_tail():
        pltpu.sync_copy(wbuf.at[0].at[pl.ds(0, _NTL)],
                        out_sh.at[pl.ds(_NBLK * 128, _NTL)])
    plsc.subcore_barrier()

    nch = (_NCH - w + _NSUB - 1) // _NSUB

    def p2(ic, _):
        ci = w + ic * _NSUB
        q = lax.div(ci, _G1 // _G)
        r2 = lax.rem(ci, _G1 // _G) * _G
        c0 = pltpu.async_copy(src_hbm.at[q, pl.ds(r2, _G)], src3, sem)
        c1 = pltpu.async_copy(dst_hbm.at[q, pl.ds(r2, _G)], dst3, sem)
        c2 = pltpu.async_copy(exb.at[c, q, pl.ds(r2, _G)], ex3, sem)
        c0.wait(); c1.wait(); c2.wait()
        dcps = []
        for g in range(_G):
            dcps.append(pltpu.async_copy(dhb.at[c].at[dst3.at[g]],
                                         den3.at[g], sem))
        zcp = [pltpu.async_copy(zp_hbm.at[c].at[src3.at[0]], wbuf.at[0], sem)]
        for cp in dcps:
            cp.wait()
        for g in range(_G):
            b = g % 2
            if g + 1 < _G:
                zcp.append(pltpu.async_copy(zp_hbm.at[c].at[src3.at[g + 1]],
                                            wbuf.at[(g + 1) % 2], sem))
            zcp[g].wait()
            gs = jnp.full((16,), g, jnp.int32)

            def agrp(j, _, gs=gs):
                row = j * 8 + half
                ex = plsc.load_gather(ex3, [gs, row, par])
                dn = plsc.load_gather(den3, [gs, row, par])
                plsc.store_scatter(ex3, [gs, row, par], ex / (dn + 1e-12))
                return _
            lax.fori_loop(0, 16, agrp, None)

            def pedge(r, _, b=b, gs=gs):
                for u in range(4):
                    rr = r * 4 + u
                    rs = jnp.full((16,), rr, jnp.int32)
                    a0 = plsc.load_gather(ex3,
                                          [gs, rs, jnp.zeros((16,), jnp.int32)])
                    a1 = plsc.load_gather(ex3,
                                          [gs, rs, jnp.ones((16,), jnp.int32)])
                    wbuf[b, rr, pl.ds(0, 16)] = wbuf[b, rr, pl.ds(0, 16)] * a0
                    wbuf[b, rr, pl.ds(16, 16)] = wbuf[b, rr, pl.ds(16, 16)] * a1
                return _
            lax.fori_loop(0, 32, pedge, None)
            pltpu.sync_copy(wbuf.at[b], out_sh.at[dst3.at[g]], add=True)
        return _
    lax.fori_loop(0, nch, p2, None)
    plsc.subcore_barrier()

    def wblk(i, _):
        r = (w + i * _NSUB) * 128
        pltpu.sync_copy(out_sh.at[pl.ds(r, 128)], outp.at[c, pl.ds(r, 128)])
        return _
    lax.fori_loop(0, nblk, wblk, None)

    @pl.when(w == _NSUB - 1)
    def _wb_tail():
        pltpu.sync_copy(out_sh.at[pl.ds(_NBLK * 128, _NTL)],
                        outp.at[c, pl.ds(_NBLK * 128, _NTL)])


_SC_PARAMS = pltpu.CompilerParams(
    needs_layout_passes=False, use_tc_tiling_on_sc=False)

_p1_call = pl.kernel(
    _p1_body,
    out_type=[
        jax.ShapeDtypeStruct((2, _NCH1, _G1, 128, 2), jnp.float32),
        jax.ShapeDtypeStruct((2, N, 8), jnp.float32),
    ],
    mesh=plsc.VectorSubcoreMesh(core_axis_name="c", subcore_axis_name="s"),
    compiler_params=_SC_PARAMS,
    scratch_types=[
        pltpu.VMEM((_G1, 128), jnp.int32),
        pltpu.VMEM((_G1, 128), jnp.int32),
        pltpu.VMEM((_K1,), jnp.int32),
        pltpu.VMEM((_G1, 128, 2 * H), jnp.float32),
        pltpu.VMEM((_G1, 128, 2 * H), jnp.float32),
        pltpu.VMEM((_G1, 128, 2), jnp.float32),
        pltpu.VMEM((_G1, 128, 16), jnp.float32),
        pltpu.VMEM((NET, H), jnp.float32),
        pltpu.SemaphoreType.DMA,
        pltpu.VMEM_SHARED((N, 16), jnp.float32),
    ],
)

_p2_call = pl.kernel(
    _p2_body,
    out_type=jax.ShapeDtypeStruct((2, N, 32), jnp.float32),
    mesh=plsc.VectorSubcoreMesh(core_axis_name="c", subcore_axis_name="s"),
    compiler_params=_SC_PARAMS,
    scratch_types=[
        pltpu.VMEM((_G, 128), jnp.int32),
        pltpu.VMEM((_G, 128), jnp.int32),
        pltpu.VMEM((_G, 128, 2), jnp.float32),
        pltpu.VMEM((_G, 128, 8), jnp.float32),
        pltpu.VMEM((2, 128, 32), jnp.float32),
        pltpu.SemaphoreType.DMA,
        pltpu.VMEM_SHARED((N, 32), jnp.float32),
    ],
)


def _edge_call(sv, zp, aet, srcR, dstR, efR):
    exb, dhb = _p1_call(sv, aet, srcR, dstR, efR)
    return _p2_call(zp, exb, dhb, srcR, dstR)


def _amat(asrc, adst):
    rows = (jnp.arange(H)[:, None] * DH + jnp.arange(DH)).reshape(-1)
    cols = jnp.repeat(jnp.arange(H), DH)
    a = jnp.zeros((COM, 2 * H), jnp.float32)
    a = a.at[rows, cols].set(asrc.reshape(-1))
    a = a.at[rows, H + cols].set(adst.reshape(-1))
    return a


def kernel(features_0, features_1, edge_index, e_feat, Wi0, bi0, Wi1, bi1,
           Wt00, bt00, Wt01, bt01, Wt10, bt10, Wt11, bt11,
           Wz0, asrc0, adst0, aet0, Wc0, bc0,
           Wz1, asrc1, adst1, aet1, Wc1, bc1,
           Wp, bp):
    srcR = edge_index[0].reshape(_NCH1, _G1, 128)
    dstR = edge_index[1].reshape(_NCH1, _G1, 128)
    efR = e_feat.reshape(_NCH1, _K1)

    h = jnp.concatenate([_mm(features_0, Wi0, bi0),
                         _mm(features_1, Wi1, bi1)], 0)
    zp0, sv0 = _zsv(h, Wz0, _amat(asrc0, adst0))
    outp0 = _edge_call(sv0, zp0, aet0, srcR, dstR, efR)
    com1, h2 = _ch(outp0, h, jnp.stack([Wt00, Wt10]),
                   jnp.stack([bt00, bt10]), bc0)
    zp1, sv1 = _zsv(h2, Wz1, _amat(asrc1, adst1))
    outp1 = _edge_call(sv1, zp1, aet1, srcR, dstR, efR)
    return _fin(outp1, h2, com1, Wc1, bc1, jnp.stack([Wt01, Wt11]),
                jnp.stack([bt01, bt11]), Wp, bp)


# speedup vs baseline: 1.0053x; 1.0053x over previous
import functools

import jax
import jax.numpy as jnp
from jax import lax
from jax.experimental import pallas as pl
from jax.experimental.pallas import tpu as pltpu
from jax.experimental.pallas import tpu_sc as plsc

N0 = 25000
N1 = 25000
N = N0 + N1
E = 800000
D_IN = 128
HID = 64
COM = 64
H = 4
DH = COM // H
NET = 4
NCLS = 16
NEG = 0.2

_BLK = 1000
_G = 5
_K = _G * 128
_NCH = E // _K
_G1 = 10
_K1 = _G1 * 128
_NCH1 = E // _K1
_NSUB = 16
_NBLK = N // 128
_NTL = N - _NBLK * 128


def _mm_body(x_ref, w_ref, b_ref, o_ref, *, act):
    y = jnp.dot(x_ref[...], w_ref[...], preferred_element_type=jnp.float32)
    y = y + b_ref[...]
    if act == "relu":
        y = jnp.maximum(y, 0.0)
    o_ref[...] = y


def _mm(x, w, b, act=None, blk=_BLK):
    m, k = x.shape
    n = w.shape[1]
    return pl.pallas_call(
        functools.partial(_mm_body, act=act),
        grid=(m // blk,),
        in_specs=[
            pl.BlockSpec((blk, k), lambda i: (i, 0)),
            pl.BlockSpec((k, n), lambda i: (0, 0)),
            pl.BlockSpec((n,), lambda i: (0,)),
        ],
        out_specs=pl.BlockSpec((blk, n), lambda i: (i, 0)),
        out_shape=jax.ShapeDtypeStruct((m, n), jnp.float32),
    )(x, w, b)


def _zsv_body(h_ref, wz_ref, a_ref, zp_ref, sv_ref):
    z = jnp.dot(h_ref[...], wz_ref[...], preferred_element_type=jnp.float32)
    zp_ref[0] = z[:, :32]
    zp_ref[1] = z[:, 32:]
    sv_ref[...] = jnp.dot(z, a_ref[...], preferred_element_type=jnp.float32)


def _zsv(h, wz, amat):
    return pl.pallas_call(
        _zsv_body,
        grid=(N // _BLK,),
        in_specs=[
            pl.BlockSpec((_BLK, HID), lambda i: (i, 0)),
            pl.BlockSpec((HID, COM), lambda i: (0, 0)),
            pl.BlockSpec((COM, 2 * H), lambda i: (0, 0)),
        ],
        out_specs=[
            pl.BlockSpec((2, _BLK, 32), lambda i: (0, i, 0)),
            pl.BlockSpec((_BLK, 2 * H), lambda i: (i, 0)),
        ],
        out_shape=[
            jax.ShapeDtypeStruct((2, N, 32), jnp.float32),
            jax.ShapeDtypeStruct((N, 2 * H), jnp.float32),
        ],
    )(h, wz, amat)


def _ch_body(op_ref, h_ref, wt_ref, bt_ref, bc_ref, com_ref, h2_ref):
    agg = jnp.concatenate([op_ref[0], op_ref[1]], axis=1)
    com_ref[...] = jnp.maximum(agg + bc_ref[...], 0.0)
    t = lax.div(pl.program_id(0), N0 // _BLK)
    y = jnp.dot(h_ref[...], wt_ref[0], preferred_element_type=jnp.float32)
    h2_ref[...] = jnp.maximum(y + bt_ref[pl.ds(t, 1), :], 0.0)


def _ch(outp, h, wt2, bt2, bc0):
    return pl.pallas_call(
        _ch_body,
        grid=(N // _BLK,),
        in_specs=[
            pl.BlockSpec((2, _BLK, 32), lambda i: (0, i, 0)),
            pl.BlockSpec((_BLK, HID), lambda i: (i, 0)),
            pl.BlockSpec((1, HID, HID), lambda i: (lax.div(i, N0 // _BLK), 0, 0)),
            pl.BlockSpec((2, HID), lambda i: (0, 0)),
            pl.BlockSpec((COM,), lambda i: (0,)),
        ],
        out_specs=[
            pl.BlockSpec((_BLK, COM), lambda i: (i, 0)),
            pl.BlockSpec((_BLK, HID), lambda i: (i, 0)),
        ],
        out_shape=[
            jax.ShapeDtypeStruct((N, COM), jnp.float32),
            jax.ShapeDtypeStruct((N, HID), jnp.float32),
        ],
    )(outp, h, wt2, bt2, bc0)


def _fin_body(op_ref, h2_ref, com_ref, wc_ref, bc_ref, wt_ref, bt_ref,
              wp_ref, bp_ref, o_ref):
    agg = jnp.concatenate([op_ref[0], op_ref[1]], axis=1)
    cm = jnp.dot(com_ref[...], wc_ref[...], preferred_element_type=jnp.float32)
    com2 = jnp.maximum(agg + cm + bc_ref[...], 0.0)
    t = lax.div(pl.program_id(0), N0 // _BLK)
    y = jnp.dot(h2_ref[...], wt_ref[0], preferred_element_type=jnp.float32)
    h3 = jnp.maximum(y + bt_ref[pl.ds(t, 1), :], 0.0)
    lg = (jnp.dot(h3, wp_ref[:HID], preferred_element_type=jnp.float32)
          + jnp.dot(com2, wp_ref[HID:], preferred_element_type=jnp.float32)
          + bp_ref[...])
    nrm = jnp.sqrt(jnp.sum(lg * lg, axis=1, keepdims=True))
    o_ref[...] = lg / jnp.maximum(nrm, 1e-12)


def _fin(outp1, h2, com1, wc1, bc1, wt2, bt2, wp, bp):
    return pl.pallas_call(
        _fin_body,
        grid=(N // _BLK,),
        in_specs=[
            pl.BlockSpec((2, _BLK, 32), lambda i: (0, i, 0)),
            pl.BlockSpec((_BLK, HID), lambda i: (i, 0)),
            pl.BlockSpec((_BLK, COM), lambda i: (i, 0)),
            pl.BlockSpec((COM, COM), lambda i: (0, 0)),
            pl.BlockSpec((COM,), lambda i: (0,)),
            pl.BlockSpec((1, HID, HID), lambda i: (lax.div(i, N0 // _BLK), 0, 0)),
            pl.BlockSpec((2, HID), lambda i: (0, 0)),
            pl.BlockSpec((HID + COM, NCLS), lambda i: (0, 0)),
            pl.BlockSpec((NCLS,), lambda i: (0,)),
        ],
        out_specs=pl.BlockSpec((_BLK, NCLS), lambda i: (i, 0)),
        out_shape=jax.ShapeDtypeStruct((N, NCLS), jnp.float32),
    )(outp1, h2, com1, wc1, bc1, wt2, bt2, wp, bp)


def _p1_body(sv_hbm, aet_hbm, src_hbm, dst_hbm, ef_hbm,
             exb, dhb,
             src3, dst3, efv, ssv, sdv, ex3, exw, aetv, sem, den_sh):
    c = lax.axis_index("c")
    w = lax.axis_index("s")
    io = lax.iota(jnp.int32, 16)
    half = lax.shift_right_logical(io, 1)
    par = lax.bitwise_and(io, 1)
    hsel = 2 * c + par
    zeros16 = jnp.zeros((16,), jnp.float32)

    for g in range(_G1):
        def _zw(r, _, g=g):
            exw[g, r, pl.ds(0, 16)] = zeros16
            return _
        lax.fori_loop(0, 128, _zw, None)

    nblk = (_NBLK - w + _NSUB - 1) // _NSUB

    def zblk(i, _):
        r = (w + i * _NSUB) * 128
        pltpu.sync_copy(exw.at[0], den_sh.at[pl.ds(r, 128)])
        return _
    lax.fori_loop(0, nblk, zblk, None)

    @pl.when(w == _NSUB - 1)
    def _zero_tail():
        pltpu.sync_copy(exw.at[0].at[pl.ds(0, _NTL)],
                        den_sh.at[pl.ds(_NBLK * 128, _NTL)])
    pltpu.sync_copy(aet_hbm, aetv)
    plsc.subcore_barrier()

    nch = (_NCH1 - w + _NSUB - 1) // _NSUB

    def p1(ic, _):
        ci = w + ic * _NSUB
        c0 = pltpu.async_copy(src_hbm.at[ci], src3, sem)
        c1 = pltpu.async_copy(dst_hbm.at[ci], dst3, sem)
        c2 = pltpu.async_copy(ef_hbm.at[ci], efv, sem)
        c0.wait(); c1.wait(); c2.wait()
        gcps = []
        for g in range(_G1):
            gcps.append(pltpu.async_copy(sv_hbm.at[src3.at[g]], ssv.at[g], sem))
            gcps.append(pltpu.async_copy(sv_hbm.at[dst3.at[g]], sdv.at[g], sem))
        for cp in gcps:
            cp.wait()
        for g in range(_G1):
            gs = jnp.full((16,), g, jnp.int32)

            def cgrp(j, _, gs=gs, g=g):
                row = j * 8 + half
                s1 = plsc.load_gather(ssv, [gs, row, hsel])
                s2 = plsc.load_gather(sdv, [gs, row, 4 + hsel])
                efl = plsc.load_gather(efv, [g * 128 + row])
                s3 = plsc.load_gather(aetv, [efl, hsel])
                lg = s1 + s2 + s3
                lg = jnp.maximum(lg, NEG * lg)
                ex = jnp.exp(lg)
                plsc.store_scatter(ex3, [gs, row, par], ex)
                plsc.store_scatter(exw, [gs, row, par], ex)
                return _
            lax.fori_loop(0, 16, cgrp, None)
        cpo = pltpu.async_copy(ex3, exb.at[c, ci], sem)
        for g in range(_G1):
            pltpu.sync_copy(exw.at[g], den_sh.at[dst3.at[g]], add=True)
        cpo.wait()
        return _
    lax.fori_loop(0, nch, p1, None)
    plsc.subcore_barrier()

    def dblk(i, _):
        r = (w + i * _NSUB) * 128
        pltpu.sync_copy(den_sh.at[pl.ds(r, 128), pl.ds(0, 8)],
                        dhb.at[c, pl.ds(r, 128)])
        return _
    lax.fori_loop(0, nblk, dblk, None)

    @pl.when(w == _NSUB - 1)
    def _den_tail():
        pltpu.sync_copy(den_sh.at[pl.ds(_NBLK * 128, _NTL), pl.ds(0, 8)],
                        dhb.at[c, pl.ds(_NBLK * 128, _NTL)])


def _p2_body(zp_hbm, exb, dhb, src_hbm, dst_hbm,
             outp,
             src3, dst3, ex3, den3, wbuf, sem, out_sh):
    c = lax.axis_index("c")
    w = lax.axis_index("s")
    io = lax.iota(jnp.int32, 16)
    half = lax.shift_right_logical(io, 1)
    par = lax.bitwise_and(io, 1)
    zeros16 = jnp.zeros((16,), jnp.float32)

    for g in range(2):
        def _zw(r, _, g=g):
            wbuf[g, r, pl.ds(0, 16)] = zeros16
            wbuf[g, r, pl.ds(16, 16)] = zeros16
            return _
        lax.fori_loop(0, 128, _zw, None)

    nblk = (_NBLK - w + _NSUB - 1) // _NSUB

    def zblk(i, _):
        r = (w + i * _NSUB) * 128
        pltpu.sync_copy(wbuf.at[0], out_sh.at[pl.ds(r, 128)])
        return _
    lax.fori_loop(0, nblk, zblk, None)

    @pl.when(w == _NSUB - 1)
    def _zero_tail():
        pltpu.sync_copy(wbuf.at[0].at[pl.ds(0, _NTL)],
                        out_sh.at[pl.ds(_NBLK * 128, _NTL)])
    plsc.subcore_barrier()

    nch = (_NCH - w + _NSUB - 1) // _NSUB

    def p2(ic, _):
        ci = w + ic * _NSUB
        q = lax.div(ci, _G1 // _G)
        r2 = lax.rem(ci, _G1 // _G) * _G
        c0 = pltpu.async_copy(src_hbm.at[q, pl.ds(r2, _G)], src3, sem)
        c1 = pltpu.async_copy(dst_hbm.at[q, pl.ds(r2, _G)], dst3, sem)
        c2 = pltpu.async_copy(exb.at[c, q, pl.ds(r2, _G)], ex3, sem)
        c0.wait(); c1.wait(); c2.wait()
        dcps = []
        for g in range(_G):
            dcps.append(pltpu.async_copy(dhb.at[c].at[dst3.at[g]],
                                         den3.at[g], sem))
        zcp = [pltpu.async_copy(zp_hbm.at[c].at[src3.at[0]], wbuf.at[0], sem)]
        for cp in dcps:
            cp.wait()
        for g in range(_G):
            b = g % 2
            if g + 1 < _G:
                zcp.append(pltpu.async_copy(zp_hbm.at[c].at[src3.at[g + 1]],
                                            wbuf.at[(g + 1) % 2], sem))
            zcp[g].wait()
            gs = jnp.full((16,), g, jnp.int32)

            def agrp(j, _, gs=gs):
                row = j * 8 + half
                ex = plsc.load_gather(ex3, [gs, row, par])
                dn = plsc.load_gather(den3, [gs, row, par])
                plsc.store_scatter(ex3, [gs, row, par], ex / (dn + 1e-12))
                return _
            lax.fori_loop(0, 16, agrp, None)

            def pedge(r, _, b=b, gs=gs):
                rs = jnp.full((16,), r, jnp.int32)
                a0 = plsc.load_gather(ex3, [gs, rs, jnp.zeros((16,), jnp.int32)])
                a1 = plsc.load_gather(ex3, [gs, rs, jnp.ones((16,), jnp.int32)])
                wbuf[b, r, pl.ds(0, 16)] = wbuf[b, r, pl.ds(0, 16)] * a0
                wbuf[b, r, pl.ds(16, 16)] = wbuf[b, r, pl.ds(16, 16)] * a1
                return _
            lax.fori_loop(0, 128, pedge, None)
            pltpu.sync_copy(wbuf.at[b], out_sh.at[dst3.at[g]], add=True)
        return _
    lax.fori_loop(0, nch, p2, None)
    plsc.subcore_barrier()

    def wblk(i, _):
        r = (w + i * _NSUB) * 128
        pltpu.sync_copy(out_sh.at[pl.ds(r, 128)], outp.at[c, pl.ds(r, 128)])
        return _
    lax.fori_loop(0, nblk, wblk, None)

    @pl.when(w == _NSUB - 1)
    def _wb_tail():
        pltpu.sync_copy(out_sh.at[pl.ds(_NBLK * 128, _NTL)],
                        outp.at[c, pl.ds(_NBLK * 128, _NTL)])


_SC_PARAMS = pltpu.CompilerParams(
    needs_layout_passes=False, use_tc_tiling_on_sc=False)

_p1_call = pl.kernel(
    _p1_body,
    out_type=[
        jax.ShapeDtypeStruct((2, _NCH1, _G1, 128, 2), jnp.float32),
        jax.ShapeDtypeStruct((2, N, 8), jnp.float32),
    ],
    mesh=plsc.VectorSubcoreMesh(core_axis_name="c", subcore_axis_name="s"),
    compiler_params=_SC_PARAMS,
    scratch_types=[
        pltpu.VMEM((_G1, 128), jnp.int32),
        pltpu.VMEM((_G1, 128), jnp.int32),
        pltpu.VMEM((_K1,), jnp.int32),
        pltpu.VMEM((_G1, 128, 2 * H), jnp.float32),
        pltpu.VMEM((_G1, 128, 2 * H), jnp.float32),
        pltpu.VMEM((_G1, 128, 2), jnp.float32),
        pltpu.VMEM((_G1, 128, 16), jnp.float32),
        pltpu.VMEM((NET, H), jnp.float32),
        pltpu.SemaphoreType.DMA,
        pltpu.VMEM_SHARED((N, 16), jnp.float32),
    ],
)

_p2_call = pl.kernel(
    _p2_body,
    out_type=jax.ShapeDtypeStruct((2, N, 32), jnp.float32),
    mesh=plsc.VectorSubcoreMesh(core_axis_name="c", subcore_axis_name="s"),
    compiler_params=_SC_PARAMS,
    scratch_types=[
        pltpu.VMEM((_G, 128), jnp.int32),
        pltpu.VMEM((_G, 128), jnp.int32),
        pltpu.VMEM((_G, 128, 2), jnp.float32),
        pltpu.VMEM((_G, 128, 8), jnp.float32),
        pltpu.VMEM((2, 128, 32), jnp.float32),
        pltpu.SemaphoreType.DMA,
        pltpu.VMEM_SHARED((N, 32), jnp.float32),
    ],
)


def _edge_call(sv, zp, aet, srcR, dstR, efR):
    exb, dhb = _p1_call(sv, aet, srcR, dstR, efR)
    return _p2_call(zp, exb, dhb, srcR, dstR)


def _amat(asrc, adst):
    rows = (jnp.arange(H)[:, None] * DH + jnp.arange(DH)).reshape(-1)
    cols = jnp.repeat(jnp.arange(H), DH)
    a = jnp.zeros((COM, 2 * H), jnp.float32)
    a = a.at[rows, cols].set(asrc.reshape(-1))
    a = a.at[rows, H + cols].set(adst.reshape(-1))
    return a


def kernel(features_0, features_1, edge_index, e_feat, Wi0, bi0, Wi1, bi1,
           Wt00, bt00, Wt01, bt01, Wt10, bt10, Wt11, bt11,
           Wz0, asrc0, adst0, aet0, Wc0, bc0,
           Wz1, asrc1, adst1, aet1, Wc1, bc1,
           Wp, bp):
    srcR = edge_index[0].reshape(_NCH1, _G1, 128)
    dstR = edge_index[1].reshape(_NCH1, _G1, 128)
    efR = e_feat.reshape(_NCH1, _K1)

    h = jnp.concatenate([_mm(features_0, Wi0, bi0),
                         _mm(features_1, Wi1, bi1)], 0)
    zp0, sv0 = _zsv(h, Wz0, _amat(asrc0, adst0))
    outp0 = _edge_call(sv0, zp0, aet0, srcR, dstR, efR)
    com1, h2 = _ch(outp0, h, jnp.stack([Wt00, Wt10]),
                   jnp.stack([bt00, bt10]), bc0)
    zp1, sv1 = _zsv(h2, Wz1, _amat(asrc1, adst1))
    outp1 = _edge_call(sv1, zp1, aet1, srcR, dstR, efR)
    return _fin(outp1, h2, com1, Wc1, bc1, jnp.stack([Wt01, Wt11]),
                jnp.stack([bt01, bt11]), Wp, bp)
